# baseline (speedup 1.0000x reference)
import functools

import jax
import jax.numpy as jnp
from jax import lax
from jax.experimental import pallas as pl
from jax.experimental.pallas import tpu as pltpu
from jax.experimental.pallas import tpu_sc as plsc

D = 16
B_TOTAL = 4096 * 50
NC, NS = 2, 16
NW = NC * NS
B_PER_W = B_TOTAL // NW

_mesh = plsc.VectorSubcoreMesh(core_axis_name="c", subcore_axis_name="s")


@functools.partial(
    pl.kernel,
    mesh=_mesh,
    out_type=jax.ShapeDtypeStruct((B_TOTAL, D), jnp.float32),
    scratch_types=[
        pltpu.VMEM((B_PER_W,), jnp.int32),
        pltpu.VMEM((B_PER_W, D), jnp.float32),
        pltpu.SemaphoreType.DMA,
    ],
    compiler_params=pltpu.CompilerParams(use_tc_tiling_on_sc=False),
)
def _sc_gather(idx_hbm, table_hbm, out_hbm, idx_v, rows_v, sem):
    wid = lax.axis_index("s") * NC + lax.axis_index("c")
    base = wid * B_PER_W
    pltpu.sync_copy(idx_hbm.at[pl.ds(base, B_PER_W)], idx_v)
    pltpu.async_copy(table_hbm.at[idx_v], rows_v, sem).wait()
    pltpu.sync_copy(rows_v, out_hbm.at[pl.ds(base, B_PER_W)])


_SCALE_ROWS = B_TOTAL * D // 128
_SCALE_BLK = 3200


def _scale_body(sig_ref, out_ref):
    row = jnp.tile(sig_ref[0, :], 8)
    out_ref[...] = jnp.broadcast_to(row[None, :], out_ref.shape)


def _scale_bcast(sigma):
    out = pl.pallas_call(
        _scale_body,
        out_shape=jax.ShapeDtypeStruct((_SCALE_ROWS, 128), jnp.float32),
        grid=(_SCALE_ROWS // _SCALE_BLK,),
        in_specs=[pl.BlockSpec((1, D), lambda i: (0, 0))],
        out_specs=pl.BlockSpec((_SCALE_BLK, 128), lambda i: (i, 0)),
    )(sigma.reshape(1, D))
    return out


def kernel(x, u, t, means, sigma):
    idx = x.reshape(-1).astype(jnp.int32)
    loc = _sc_gather(idx, means).reshape(x.shape + (D,))
    scale = _scale_bcast(sigma).reshape(x.shape + (D,))
    return (loc, scale)
